# Initial kernel scaffold; baseline (speedup 1.0000x reference)
import jax
import jax.numpy as jnp
from jax.experimental import pallas as pl


def kernel(xA, edge_indexA, edge_attrA, batchA, xB, edge_indexB, edge_attrB, batchB, context, params):
    raise NotImplementedError("write your pallas kernel here")


# speedup vs baseline: 10.8339x; 10.8339x over previous
import functools

import jax
import jax.numpy as jnp
from jax import lax
from jax.experimental import pallas as pl
from jax.experimental.pallas import tpu as pltpu
from jax.experimental.pallas import tpu_sc as plsc

N = 10000
E = 320000
G = 256
EMB = 128
NPAD = 10240
NSUB = 16
CHUNK = 128
KW = 160
IB = 16
EPW = KW * CHUNK
EP = EPW * NSUB
CHUNKS = EP // CHUNK
ZR = NPAD // NSUB
REP = 128

HIGH = lax.Precision.HIGHEST


def _dot(a, b):
    return lax.dot_general(a, b, (((a.ndim - 1,), (0,)), ((), ())),
                           precision=HIGH, preferred_element_type=jnp.float32)


def _dotd(a, b):
    return lax.dot_general(a, b, (((a.ndim - 1,), (0,)), ((), ())),
                           preferred_element_type=jnp.float32)


def _dotT(a, b):
    return lax.dot_general(a, b, (((0,), (0,)), ((), ())),
                           precision=HIGH, preferred_element_type=jnp.float32)


def _sc_segment_rows(tab2, src2, dst2, zeros, d):
    mesh = plsc.VectorSubcoreMesh(core_axis_name="c", subcore_axis_name="s")

    @functools.partial(
        pl.kernel, mesh=mesh,
        out_type=jax.ShapeDtypeStruct((2, NPAD, d), jnp.float32),
        scratch_types=[
            pltpu.VMEM((IB * CHUNK,), jnp.int32),
            pltpu.VMEM((IB, CHUNK), jnp.int32),
            pltpu.VMEM((CHUNK, d), jnp.float32),
            pltpu.VMEM((CHUNK, d), jnp.float32),
            pltpu.VMEM_SHARED((NPAD, d), jnp.float32),
            pltpu.SemaphoreType.DMA,
            pltpu.SemaphoreType.DMA,
        ])
    def scatter_kernel(tab_h, src_h, dst_h, z_h, out_h,
                       src_v, dst_v, buf0, buf1, acc, sem0, sem1):
        c = lax.axis_index("c")
        s = lax.axis_index("s")
        pltpu.sync_copy(z_h.at[pl.ds(s * ZR, ZR)], acc.at[pl.ds(s * ZR, ZR)])
        plsc.subcore_barrier()

        def run(tab, src, dst):
            def blk(ib, carry):
                pltpu.sync_copy(
                    src.at[pl.ds(s * EPW + ib * (IB * CHUNK), IB * CHUNK)],
                    src_v)
                pltpu.sync_copy(dst.at[pl.ds(s * KW + ib * IB, IB)], dst_v)

                def pair(j, c2):
                    g0 = pltpu.async_copy(
                        tab.at[src_v.at[pl.ds((2 * j) * CHUNK, CHUNK)]], buf0,
                        sem0)
                    g1 = pltpu.async_copy(
                        tab.at[src_v.at[pl.ds((2 * j + 1) * CHUNK, CHUNK)]],
                        buf1, sem1)
                    g0.wait()
                    pltpu.sync_copy(buf0, acc.at[dst_v.at[2 * j]], add=True)
                    g1.wait()
                    pltpu.sync_copy(buf1, acc.at[dst_v.at[2 * j + 1]],
                                    add=True)
                    return c2

                lax.fori_loop(0, IB // 2, pair, 0)
                return carry

            lax.fori_loop(0, KW // IB, blk, 0)

        @pl.when(c == 0)
        def _():
            run(tab_h.at[0], src_h.at[0], dst_h.at[0])

        @pl.when(c == 1)
        def _():
            run(tab_h.at[1], src_h.at[1], dst_h.at[1])

        plsc.subcore_barrier()
        pltpu.sync_copy(acc.at[pl.ds(s * ZR, ZR)],
                        out_h.at[c].at[pl.ds(s * ZR, ZR)])

    return scatter_kernel(tab2, src2, dst2, zeros)


EMB_BLK = 2000


def _tc_embed(x0, x1, emb1p, emb2p):
    def body(x0_r, x1_r, e1_r, e2_r, h_r):
        oh1 = (x0_r[0] == lax.broadcasted_iota(jnp.int32, (EMB_BLK, 128), 1)
               ).astype(jnp.float32)
        oh2 = (x1_r[0] == lax.broadcasted_iota(jnp.int32, (EMB_BLK, 128), 1)
               ).astype(jnp.float32)
        h_r[0] = _dot(oh1, e1_r[...]) + _dot(oh2, e2_r[...])

    full = lambda shape: pl.BlockSpec(shape, lambda g, b: (0,) * len(shape))
    return pl.pallas_call(
        body, grid=(2, N // EMB_BLK),
        in_specs=[
            pl.BlockSpec((1, EMB_BLK, 1), lambda g, b: (g, b, 0)),
            pl.BlockSpec((1, EMB_BLK, 1), lambda g, b: (g, b, 0)),
            full((128, EMB)), full((128, EMB)),
        ],
        out_specs=pl.BlockSpec((1, EMB_BLK, EMB), lambda g, b: (g, b, 0)),
        out_shape=jax.ShapeDtypeStruct((2, N, EMB), jnp.float32),
    )(x0, x1, emb1p, emb2p)


def _tc_ctx(context, ceW1, ceb1, ceW2, ceb2):
    def body(ctx_r, w1_r, b1_r, w2_r, b2_r, c_r):
        t = jnp.maximum(_dotd(ctx_r[...], w1_r[...]) + b1_r[...], 0.0)
        c_r[...] = _dotd(t, w2_r[...]) + b2_r[...]

    return pl.pallas_call(
        body,
        out_shape=jax.ShapeDtypeStruct((G, EMB), jnp.float32),
    )(context, ceW1, ceb1, ceW2, ceb2)


GINE_T = 1000


def _tc_gine(aggr, h, cnt, EE, cself, W1, b1, W2, b2, bng, bnb, do_relu):
    def body(a_r, h_r, c_r, ee_r, cs_r, w1_r, b1_r, w2_r, b2_r, g_r, bb_r,
             o_r, t_s):
        nt = N // GINE_T

        def p1(i, s1):
            sl = pl.ds(i * GINE_T, GINE_T)
            a = (a_r[0, sl, :] + h_r[0, sl, :] + cs_r[...]
                 + _dot(c_r[0, sl, :], ee_r[...]))
            t = jnp.maximum(_dotd(a, w1_r[...]) + b1_r[...], 0.0)
            t = _dotd(t, w2_r[...]) + b2_r[...]
            t_s[sl, :] = t
            return s1 + jnp.sum(t, axis=0, keepdims=True)

        s1 = lax.fori_loop(0, nt, p1, jnp.zeros((1, EMB), jnp.float32))
        m = s1 * (1.0 / N)

        def p2(i, s2):
            sl = pl.ds(i * GINE_T, GINE_T)
            d = t_s[sl, :] - m
            return s2 + jnp.sum(d * d, axis=0, keepdims=True)

        s2 = lax.fori_loop(0, nt, p2, jnp.zeros((1, EMB), jnp.float32))
        v = s2 * (1.0 / N)
        sc = lax.rsqrt(v + 1e-5) * g_r[...]
        sh = bb_r[...] - m * sc

        def p3(i, c):
            sl = pl.ds(i * GINE_T, GINE_T)
            hn = t_s[sl, :] * sc + sh
            if do_relu:
                hn = jnp.maximum(hn, 0.0)
            o_r[0, sl, :] = hn
            return c

        lax.fori_loop(0, nt, p3, 0)

    full = lambda shape: pl.BlockSpec(shape, lambda g: (0,) * len(shape))
    return pl.pallas_call(
        body, grid=(2,),
        in_specs=[
            pl.BlockSpec((1, NPAD, EMB), lambda g: (g, 0, 0)),
            pl.BlockSpec((1, N, EMB), lambda g: (g, 0, 0)),
            pl.BlockSpec((1, N, 16), lambda g: (g, 0, 0)),
            full((16, EMB)), full((1, EMB)),
            full((EMB, 2 * EMB)), full((1, 2 * EMB)),
            full((2 * EMB, EMB)), full((1, EMB)),
            full((1, EMB)), full((1, EMB)),
        ],
        out_specs=pl.BlockSpec((1, N, EMB), lambda g: (g, 0, 0)),
        out_shape=jax.ShapeDtypeStruct((2, N, EMB), jnp.float32),
        scratch_shapes=[pltpu.VMEM((N, EMB), jnp.float32)],
    )(aggr, h, cnt, EE, cself, W1, b1, W2, b2, bng, bnb)


GAT_T = 1000


def _tc_gat1(h, ctx, b2, Ws, Wd, a_s, a_d, gatb):
    def body(h_r, ctx_r, b_r, ws_r, wd_r, as_r, ad_r, gb_r, o_r, hs_s, e_s):
        nt = N // GAT_T

        def onehot(i):
            sl = pl.ds(i * GAT_T, GAT_T)
            return (b_r[0, sl, :]
                    == lax.broadcasted_iota(jnp.int32, (GAT_T, G), 1)
                    ).astype(jnp.float32)

        hd = _dotd(ctx_r[...], wd_r[...])
        ed = _dotd(hd, ad_r[...])

        def p1(i, M):
            sl = pl.ds(i * GAT_T, GAT_T)
            O = onehot(i)
            hs = _dotd(h_r[0, sl, :], ws_r[...])
            hs_s[sl, :] = hs
            e = _dotd(hs, as_r[...]) + _dot(O, ed)
            e = jnp.where(e > 0, e, 0.2 * e)
            e_s[sl, :] = e
            return jnp.maximum(
                M, jnp.max(e - 1e30 * (1.0 - O), axis=0, keepdims=True))

        M = lax.fori_loop(0, nt, p1, jnp.full((1, G), -1e30, jnp.float32))
        Mc = jnp.reshape(M, (G, 1))

        def p2(i, ssum):
            sl = pl.ds(i * GAT_T, GAT_T)
            O = onehot(i)
            ex = jnp.exp(e_s[sl, :] - _dot(O, Mc))
            return ssum + _dotT(O, ex)

        ssum = lax.fori_loop(0, nt, p2, jnp.zeros((G, 1), jnp.float32))

        def p3(i, acc):
            sl = pl.ds(i * GAT_T, GAT_T)
            O = onehot(i)
            ex = jnp.exp(e_s[sl, :] - _dot(O, Mc))
            alpha = ex / (_dot(O, ssum) + 1e-16)
            return acc + _dotT(O, alpha * hs_s[sl, :])

        acc = lax.fori_loop(0, nt, p3, jnp.zeros((G, EMB), jnp.float32))
        o_r[0] = acc + gb_r[...]

    full = lambda shape: pl.BlockSpec(shape, lambda g: (0,) * len(shape))
    return pl.pallas_call(
        body, grid=(2,),
        in_specs=[
            pl.BlockSpec((1, N, EMB), lambda g: (g, 0, 0)),
            full((G, EMB)),
            pl.BlockSpec((1, N, 1), lambda g: (g, 0, 0)),
            full((EMB, EMB)), full((EMB, EMB)),
            full((EMB, 1)), full((EMB, 1)), full((1, EMB)),
        ],
        out_specs=pl.BlockSpec((1, G, EMB), lambda g: (g, 0, 0)),
        out_shape=jax.ShapeDtypeStruct((2, G, EMB), jnp.float32),
        scratch_shapes=[pltpu.VMEM((N, EMB), jnp.float32),
                        pltpu.VMEM((N, 1), jnp.float32)],
    )(h, ctx, b2, Ws, Wd, a_s, a_d, gatb)


def _tc_gat2(ctx, gat2, injW, injb):
    def body(ctx_r, g_r, w_r, b_r, c_r, i_r):
        cn = ctx_r[...] + g_r[0] + g_r[1]
        c_r[...] = cn
        i_r[...] = _dotd(cn, w_r[...]) + b_r[...]

    return pl.pallas_call(
        body,
        out_shape=(jax.ShapeDtypeStruct((G, EMB), jnp.float32),
                   jax.ShapeDtypeStruct((G, EMB), jnp.float32)),
    )(ctx, gat2, injW, injb)


def _tc_gat3(h, b2, inj):
    def body(h_r, b_r, i_r, o_r):
        O = (b_r[0] == lax.broadcasted_iota(jnp.int32, (GAT_T, G), 1)
             ).astype(jnp.float32)
        o_r[0] = h_r[0] + _dot(O, i_r[...])

    full = lambda shape: pl.BlockSpec(shape, lambda g, b: (0,) * len(shape))
    return pl.pallas_call(
        body, grid=(2, N // GAT_T),
        in_specs=[
            pl.BlockSpec((1, GAT_T, EMB), lambda g, b: (g, b, 0)),
            pl.BlockSpec((1, GAT_T, 1), lambda g, b: (g, b, 0)),
            full((G, EMB)),
        ],
        out_specs=pl.BlockSpec((1, GAT_T, EMB), lambda g, b: (g, b, 0)),
        out_shape=jax.ShapeDtypeStruct((2, N, EMB), jnp.float32),
    )(h, b2, inj)


def _tc_final(h, ctx, b2, otW1, otb1, otW2, otb2, rsW, rsb, mws, mbs):
    def body(h_r, ctx_r, b_r, ow1, ob1, ow2, ob2, rw, rb,
             m0, mb0, m1, mb1, m2, mb2, m3, mb3, o_r):
        gs = []
        for g in range(2):
            def pool(i, carry):
                acc, cnt = carry
                sl = pl.ds(i * GAT_T, GAT_T)
                O = (b_r[g, sl, :]
                     == lax.broadcasted_iota(jnp.int32, (GAT_T, G), 1)
                     ).astype(jnp.float32)
                return (acc + _dotT(O, h_r[g, sl, :]),
                        cnt + jnp.sum(O, axis=0, keepdims=True))

            acc, cnt = lax.fori_loop(
                0, N // GAT_T, pool,
                (jnp.zeros((G, EMB), jnp.float32),
                 jnp.zeros((1, G), jnp.float32)))
            cnt = jnp.maximum(cnt, 1.0)
            gm = acc / jnp.reshape(cnt, (G, 1))
            gm = jnp.maximum(_dotd(gm, ow1[...]) + ob1[...], 0.0)
            gs.append(_dotd(gm, ow2[...]) + ob2[...])
        r = _dotd(gs[0] + gs[1], rw[...]) + rb[...]
        z = jnp.concatenate([r, ctx_r[...]], axis=1)
        for w_r, b_r2 in ((m0, mb0), (m1, mb1), (m2, mb2)):
            z = _dotd(z, w_r[...]) + b_r2[...]
            z = jnp.where(z > 0, z, 0.01 * z)
        o_r[...] = _dotd(z, m3[...]) + mb3[...]

    return pl.pallas_call(
        body,
        out_shape=jax.ShapeDtypeStruct((G, 1), jnp.float32),
    )(h, ctx, b2, otW1, otb1, otW2, otb2, rsW, rsb,
      mws[0], mbs[0], mws[1], mbs[1], mws[2], mbs[2], mws[3], mbs[3])


def _prep_graph(ei, ea):
    pad = jnp.arange(EP - E, dtype=jnp.int32)
    src = jnp.concatenate([ei[0], pad % N])
    dst = jnp.concatenate([ei[1], N + pad % (NPAD - N)]).reshape(CHUNKS, CHUNK)
    ear = jnp.arange(E, dtype=jnp.int32)
    code = ea[:, 0] * 3 + ea[:, 1]
    codeidx = jnp.concatenate(
        [code * REP + ear % REP, (pad % 15) * REP + pad % REP])
    return src, dst, codeidx


def kernel(xA, edge_indexA, edge_attrA, batchA, xB, edge_indexB, edge_attrB,
           batchB, context, params):
    p = params
    f32 = jnp.float32
    row = lambda v: v.reshape(1, -1).astype(f32)

    srcA, dstA, codeA = _prep_graph(edge_indexA, edge_attrA)
    srcB, dstB, codeB = _prep_graph(edge_indexB, edge_attrB)
    src2 = jnp.stack([srcA, srcB])
    dst2 = jnp.stack([dstA, dstB])
    code2 = jnp.stack([codeA, codeB])
    b2 = jnp.stack([batchA, batchB]).reshape(2, N, 1)
    x0 = jnp.stack([xA[:, 0], xB[:, 0]]).reshape(2, N, 1)
    x1 = jnp.stack([xA[:, 1], xB[:, 1]]).reshape(2, N, 1)

    codes = jnp.arange(15, dtype=jnp.int32)
    lanes = jnp.arange(128, dtype=jnp.int32)
    oh = ((lanes[None, :] == codes[:, None] // 3)
          | (lanes[None, :] == 5 + codes[:, None] % 3)).astype(f32)
    ohrep = jnp.repeat(oh, REP, axis=0)
    ohrep2 = jnp.stack([ohrep, ohrep])

    zerosE = jnp.zeros((NPAD, EMB), f32)

    emb1p = jnp.concatenate([p['x_emb1'], jnp.zeros((128 - 119, EMB), f32)], 0)
    emb2p = jnp.concatenate([p['x_emb2'], jnp.zeros((128 - 3, EMB), f32)], 0)

    h = _tc_embed(x0, x1, emb1p, emb2p)
    ctx = _tc_ctx(context, p['ce_W1'], row(p['ce_b1']),
                  p['ce_W2'], row(p['ce_b2']))

    cnt = _sc_segment_rows(ohrep2, code2, dst2, zerosE, 128)[:, :N, :16]

    for l in range(5):
        if l < 3:
            pre = 'basic%d_' % l
        else:
            pre = 'cf%d_' % (l - 3)
            gat2 = _tc_gat1(h, ctx, b2, p[pre + 'gat_Ws'], p[pre + 'gat_Wd'],
                            p[pre + 'gat_as'].reshape(EMB, 1),
                            p[pre + 'gat_ad'].reshape(EMB, 1),
                            row(p[pre + 'gat_b']))
            ctx, inj = _tc_gat2(ctx, gat2, p[pre + 'inj_W'],
                                row(p[pre + 'inj_b']))
            h = _tc_gat3(h, b2, inj)
        aggr = _sc_segment_rows(h, src2, dst2, zerosE, EMB)
        EE = jnp.concatenate(
            [p[pre + 'ee1'], p[pre + 'ee2'], jnp.zeros((8, EMB), f32)], 0)
        cself = (p[pre + 'ee1'][4] + p[pre + 'ee2'][0]).reshape(1, EMB)
        h = _tc_gine(aggr, h, cnt, EE, cself,
                     p[pre + 'W1'], row(p[pre + 'b1']),
                     p[pre + 'W2'], row(p[pre + 'b2']),
                     row(p[pre + 'bn_g']), row(p[pre + 'bn_b']),
                     do_relu=(l in (0, 1, 3)))

    return _tc_final(h, ctx, b2,
                     p['ot_W1'], row(p['ot_b1']), p['ot_W2'], row(p['ot_b2']),
                     p['rs_W'], row(p['rs_b']),
                     [p['mlp_W%d' % i] for i in range(4)],
                     [row(p['mlp_b%d' % i]) for i in range(4)])


# speedup vs baseline: 10.8586x; 1.0023x over previous
import functools

import jax
import jax.numpy as jnp
from jax import lax
from jax.experimental import pallas as pl
from jax.experimental.pallas import tpu as pltpu
from jax.experimental.pallas import tpu_sc as plsc

N = 10000
E = 320000
G = 256
EMB = 128
NPAD = 10240
NSUB = 16
CHUNK = 128
KW = 160
IB = 16
EPW = KW * CHUNK
EP = EPW * NSUB
CHUNKS = EP // CHUNK
ZR = NPAD // NSUB
REP = 128

HIGH = lax.Precision.HIGHEST


def _dot(a, b):
    return lax.dot_general(a, b, (((a.ndim - 1,), (0,)), ((), ())),
                           precision=HIGH, preferred_element_type=jnp.float32)


def _dotd(a, b):
    return lax.dot_general(a, b, (((a.ndim - 1,), (0,)), ((), ())),
                           preferred_element_type=jnp.float32)


def _dotT(a, b):
    return lax.dot_general(a, b, (((0,), (0,)), ((), ())),
                           precision=HIGH, preferred_element_type=jnp.float32)


def _sc_segment_rows(tab2, src2, dst2, zeros, d):
    mesh = plsc.VectorSubcoreMesh(core_axis_name="c", subcore_axis_name="s")

    @functools.partial(
        pl.kernel, mesh=mesh,
        out_type=jax.ShapeDtypeStruct((2, NPAD, d), jnp.float32),
        scratch_types=[
            pltpu.VMEM((IB * CHUNK,), jnp.int32),
            pltpu.VMEM((IB, CHUNK), jnp.int32),
            pltpu.VMEM((CHUNK, d), jnp.float32),
            pltpu.VMEM((CHUNK, d), jnp.float32),
            pltpu.VMEM_SHARED((NPAD, d), jnp.float32),
            pltpu.SemaphoreType.DMA,
            pltpu.SemaphoreType.DMA,
        ])
    def scatter_kernel(tab_h, src_h, dst_h, z_h, out_h,
                       src_v, dst_v, buf0, buf1, acc,
                       sem0, sem1):
        c = lax.axis_index("c")
        s = lax.axis_index("s")
        pltpu.sync_copy(z_h.at[pl.ds(s * ZR, ZR)], acc.at[pl.ds(s * ZR, ZR)])
        plsc.subcore_barrier()

        def run(tab, src, dst):
            def blk(ib, carry):
                pltpu.sync_copy(
                    src.at[pl.ds(s * EPW + ib * (IB * CHUNK), IB * CHUNK)],
                    src_v)
                pltpu.sync_copy(dst.at[pl.ds(s * KW + ib * IB, IB)], dst_v)

                def pair(j, c2):
                    bufs = (buf0, buf1)
                    sems = (sem0, sem1)
                    gs = []
                    for u in range(2):
                        gs.append(pltpu.async_copy(
                            tab.at[src_v.at[pl.ds((2 * j + u) * CHUNK, CHUNK)]],
                            bufs[u], sems[u]))
                    for u in range(2):
                        gs[u].wait()
                        pltpu.sync_copy(bufs[u], acc.at[dst_v.at[2 * j + u]],
                                        add=True)
                    return c2

                lax.fori_loop(0, IB // 2, pair, 0)
                return carry

            lax.fori_loop(0, KW // IB, blk, 0)

        @pl.when(c == 0)
        def _():
            run(tab_h.at[0], src_h.at[0], dst_h.at[0])

        @pl.when(c == 1)
        def _():
            run(tab_h.at[1], src_h.at[1], dst_h.at[1])

        plsc.subcore_barrier()
        pltpu.sync_copy(acc.at[pl.ds(s * ZR, ZR)],
                        out_h.at[c].at[pl.ds(s * ZR, ZR)])

    return scatter_kernel(tab2, src2, dst2, zeros)


EMB_BLK = 2000


def _tc_embed(x0, x1, emb1p, emb2p):
    def body(x0_r, x1_r, e1_r, e2_r, h_r):
        oh1 = (x0_r[0] == lax.broadcasted_iota(jnp.int32, (EMB_BLK, 128), 1)
               ).astype(jnp.float32)
        oh2 = (x1_r[0] == lax.broadcasted_iota(jnp.int32, (EMB_BLK, 128), 1)
               ).astype(jnp.float32)
        h_r[0] = _dot(oh1, e1_r[...]) + _dot(oh2, e2_r[...])

    full = lambda shape: pl.BlockSpec(shape, lambda g, b: (0,) * len(shape))
    return pl.pallas_call(
        body, grid=(2, N // EMB_BLK),
        in_specs=[
            pl.BlockSpec((1, EMB_BLK, 1), lambda g, b: (g, b, 0)),
            pl.BlockSpec((1, EMB_BLK, 1), lambda g, b: (g, b, 0)),
            full((128, EMB)), full((128, EMB)),
        ],
        out_specs=pl.BlockSpec((1, EMB_BLK, EMB), lambda g, b: (g, b, 0)),
        out_shape=jax.ShapeDtypeStruct((2, N, EMB), jnp.float32),
    )(x0, x1, emb1p, emb2p)


def _tc_ctx(context, ceW1, ceb1, ceW2, ceb2):
    def body(ctx_r, w1_r, b1_r, w2_r, b2_r, c_r):
        t = jnp.maximum(_dotd(ctx_r[...], w1_r[...]) + b1_r[...], 0.0)
        c_r[...] = _dotd(t, w2_r[...]) + b2_r[...]

    return pl.pallas_call(
        body,
        out_shape=jax.ShapeDtypeStruct((G, EMB), jnp.float32),
    )(context, ceW1, ceb1, ceW2, ceb2)


GINE_T = 1000


def _tc_gine(aggr, h, cnt, EE, cself, W1, b1, W2, b2, bng, bnb, do_relu):
    def body(a_r, h_r, c_r, ee_r, cs_r, w1_r, b1_r, w2_r, b2_r, g_r, bb_r,
             o_r, t_s):
        nt = N // GINE_T

        def p1(i, s1):
            sl = pl.ds(i * GINE_T, GINE_T)
            a = (a_r[0, sl, :] + h_r[0, sl, :] + cs_r[...]
                 + _dot(c_r[0, sl, :], ee_r[...]))
            t = jnp.maximum(_dotd(a, w1_r[...]) + b1_r[...], 0.0)
            t = _dotd(t, w2_r[...]) + b2_r[...]
            t_s[sl, :] = t
            return s1 + jnp.sum(t, axis=0, keepdims=True)

        s1 = lax.fori_loop(0, nt, p1, jnp.zeros((1, EMB), jnp.float32))
        m = s1 * (1.0 / N)

        def p2(i, s2):
            sl = pl.ds(i * GINE_T, GINE_T)
            d = t_s[sl, :] - m
            return s2 + jnp.sum(d * d, axis=0, keepdims=True)

        s2 = lax.fori_loop(0, nt, p2, jnp.zeros((1, EMB), jnp.float32))
        v = s2 * (1.0 / N)
        sc = lax.rsqrt(v + 1e-5) * g_r[...]
        sh = bb_r[...] - m * sc

        def p3(i, c):
            sl = pl.ds(i * GINE_T, GINE_T)
            hn = t_s[sl, :] * sc + sh
            if do_relu:
                hn = jnp.maximum(hn, 0.0)
            o_r[0, sl, :] = hn
            return c

        lax.fori_loop(0, nt, p3, 0)

    full = lambda shape: pl.BlockSpec(shape, lambda g: (0,) * len(shape))
    return pl.pallas_call(
        body, grid=(2,),
        in_specs=[
            pl.BlockSpec((1, NPAD, EMB), lambda g: (g, 0, 0)),
            pl.BlockSpec((1, N, EMB), lambda g: (g, 0, 0)),
            pl.BlockSpec((1, N, 16), lambda g: (g, 0, 0)),
            full((16, EMB)), full((1, EMB)),
            full((EMB, 2 * EMB)), full((1, 2 * EMB)),
            full((2 * EMB, EMB)), full((1, EMB)),
            full((1, EMB)), full((1, EMB)),
        ],
        out_specs=pl.BlockSpec((1, N, EMB), lambda g: (g, 0, 0)),
        out_shape=jax.ShapeDtypeStruct((2, N, EMB), jnp.float32),
        scratch_shapes=[pltpu.VMEM((N, EMB), jnp.float32)],
    )(aggr, h, cnt, EE, cself, W1, b1, W2, b2, bng, bnb)


GAT_T = 1000


def _tc_gat1(h, ctx, b2, Ws, Wd, a_s, a_d, gatb):
    def body(h_r, ctx_r, b_r, ws_r, wd_r, as_r, ad_r, gb_r, o_r, hs_s, e_s):
        nt = N // GAT_T

        def onehot(i):
            sl = pl.ds(i * GAT_T, GAT_T)
            return (b_r[0, sl, :]
                    == lax.broadcasted_iota(jnp.int32, (GAT_T, G), 1)
                    ).astype(jnp.float32)

        hd = _dotd(ctx_r[...], wd_r[...])
        ed = _dotd(hd, ad_r[...])

        def p1(i, M):
            sl = pl.ds(i * GAT_T, GAT_T)
            O = onehot(i)
            hs = _dotd(h_r[0, sl, :], ws_r[...])
            hs_s[sl, :] = hs
            e = _dotd(hs, as_r[...]) + _dot(O, ed)
            e = jnp.where(e > 0, e, 0.2 * e)
            e_s[sl, :] = e
            return jnp.maximum(
                M, jnp.max(e - 1e30 * (1.0 - O), axis=0, keepdims=True))

        M = lax.fori_loop(0, nt, p1, jnp.full((1, G), -1e30, jnp.float32))
        Mc = jnp.reshape(M, (G, 1))

        def p2(i, ssum):
            sl = pl.ds(i * GAT_T, GAT_T)
            O = onehot(i)
            ex = jnp.exp(e_s[sl, :] - _dot(O, Mc))
            return ssum + _dotT(O, ex)

        ssum = lax.fori_loop(0, nt, p2, jnp.zeros((G, 1), jnp.float32))

        def p3(i, acc):
            sl = pl.ds(i * GAT_T, GAT_T)
            O = onehot(i)
            ex = jnp.exp(e_s[sl, :] - _dot(O, Mc))
            alpha = ex / (_dot(O, ssum) + 1e-16)
            return acc + _dotT(O, alpha * hs_s[sl, :])

        acc = lax.fori_loop(0, nt, p3, jnp.zeros((G, EMB), jnp.float32))
        o_r[0] = acc + gb_r[...]

    full = lambda shape: pl.BlockSpec(shape, lambda g: (0,) * len(shape))
    return pl.pallas_call(
        body, grid=(2,),
        in_specs=[
            pl.BlockSpec((1, N, EMB), lambda g: (g, 0, 0)),
            full((G, EMB)),
            pl.BlockSpec((1, N, 1), lambda g: (g, 0, 0)),
            full((EMB, EMB)), full((EMB, EMB)),
            full((EMB, 1)), full((EMB, 1)), full((1, EMB)),
        ],
        out_specs=pl.BlockSpec((1, G, EMB), lambda g: (g, 0, 0)),
        out_shape=jax.ShapeDtypeStruct((2, G, EMB), jnp.float32),
        scratch_shapes=[pltpu.VMEM((N, EMB), jnp.float32),
                        pltpu.VMEM((N, 1), jnp.float32)],
    )(h, ctx, b2, Ws, Wd, a_s, a_d, gatb)


def _tc_gat2(ctx, gat2, injW, injb):
    def body(ctx_r, g_r, w_r, b_r, c_r, i_r):
        cn = ctx_r[...] + g_r[0] + g_r[1]
        c_r[...] = cn
        i_r[...] = _dotd(cn, w_r[...]) + b_r[...]

    return pl.pallas_call(
        body,
        out_shape=(jax.ShapeDtypeStruct((G, EMB), jnp.float32),
                   jax.ShapeDtypeStruct((G, EMB), jnp.float32)),
    )(ctx, gat2, injW, injb)


def _tc_gat3(h, b2, inj):
    def body(h_r, b_r, i_r, o_r):
        O = (b_r[0] == lax.broadcasted_iota(jnp.int32, (GAT_T, G), 1)
             ).astype(jnp.float32)
        o_r[0] = h_r[0] + _dot(O, i_r[...])

    full = lambda shape: pl.BlockSpec(shape, lambda g, b: (0,) * len(shape))
    return pl.pallas_call(
        body, grid=(2, N // GAT_T),
        in_specs=[
            pl.BlockSpec((1, GAT_T, EMB), lambda g, b: (g, b, 0)),
            pl.BlockSpec((1, GAT_T, 1), lambda g, b: (g, b, 0)),
            full((G, EMB)),
        ],
        out_specs=pl.BlockSpec((1, GAT_T, EMB), lambda g, b: (g, b, 0)),
        out_shape=jax.ShapeDtypeStruct((2, N, EMB), jnp.float32),
    )(h, b2, inj)


def _tc_final(h, ctx, b2, otW1, otb1, otW2, otb2, rsW, rsb, mws, mbs):
    def body(h_r, ctx_r, b_r, ow1, ob1, ow2, ob2, rw, rb,
             m0, mb0, m1, mb1, m2, mb2, m3, mb3, o_r):
        gs = []
        for g in range(2):
            def pool(i, carry):
                acc, cnt = carry
                sl = pl.ds(i * GAT_T, GAT_T)
                O = (b_r[g, sl, :]
                     == lax.broadcasted_iota(jnp.int32, (GAT_T, G), 1)
                     ).astype(jnp.float32)
                return (acc + _dotT(O, h_r[g, sl, :]),
                        cnt + jnp.sum(O, axis=0, keepdims=True))

            acc, cnt = lax.fori_loop(
                0, N // GAT_T, pool,
                (jnp.zeros((G, EMB), jnp.float32),
                 jnp.zeros((1, G), jnp.float32)))
            cnt = jnp.maximum(cnt, 1.0)
            gm = acc / jnp.reshape(cnt, (G, 1))
            gm = jnp.maximum(_dotd(gm, ow1[...]) + ob1[...], 0.0)
            gs.append(_dotd(gm, ow2[...]) + ob2[...])
        r = _dotd(gs[0] + gs[1], rw[...]) + rb[...]
        z = jnp.concatenate([r, ctx_r[...]], axis=1)
        for w_r, b_r2 in ((m0, mb0), (m1, mb1), (m2, mb2)):
            z = _dotd(z, w_r[...]) + b_r2[...]
            z = jnp.where(z > 0, z, 0.01 * z)
        o_r[...] = _dotd(z, m3[...]) + mb3[...]

    return pl.pallas_call(
        body,
        out_shape=jax.ShapeDtypeStruct((G, 1), jnp.float32),
    )(h, ctx, b2, otW1, otb1, otW2, otb2, rsW, rsb,
      mws[0], mbs[0], mws[1], mbs[1], mws[2], mbs[2], mws[3], mbs[3])


def _prep_graph(ei, ea):
    pad = jnp.arange(EP - E, dtype=jnp.int32)
    src = jnp.concatenate([ei[0], pad % N])
    dst = jnp.concatenate([ei[1], N + pad % (NPAD - N)]).reshape(CHUNKS, CHUNK)
    ear = jnp.arange(E, dtype=jnp.int32)
    code = ea[:, 0] * 3 + ea[:, 1]
    codeidx = jnp.concatenate(
        [code * REP + ear % REP, (pad % 15) * REP + pad % REP])
    return src, dst, codeidx


def kernel(xA, edge_indexA, edge_attrA, batchA, xB, edge_indexB, edge_attrB,
           batchB, context, params):
    p = params
    f32 = jnp.float32
    row = lambda v: v.reshape(1, -1).astype(f32)

    srcA, dstA, codeA = _prep_graph(edge_indexA, edge_attrA)
    srcB, dstB, codeB = _prep_graph(edge_indexB, edge_attrB)
    src2 = jnp.stack([srcA, srcB])
    dst2 = jnp.stack([dstA, dstB])
    code2 = jnp.stack([codeA, codeB])
    b2 = jnp.stack([batchA, batchB]).reshape(2, N, 1)
    x0 = jnp.stack([xA[:, 0], xB[:, 0]]).reshape(2, N, 1)
    x1 = jnp.stack([xA[:, 1], xB[:, 1]]).reshape(2, N, 1)

    codes = jnp.arange(15, dtype=jnp.int32)
    lanes = jnp.arange(128, dtype=jnp.int32)
    oh = ((lanes[None, :] == codes[:, None] // 3)
          | (lanes[None, :] == 5 + codes[:, None] % 3)).astype(f32)
    ohrep = jnp.repeat(oh, REP, axis=0)
    ohrep2 = jnp.stack([ohrep, ohrep])

    zerosE = jnp.zeros((NPAD, EMB), f32)

    emb1p = jnp.concatenate([p['x_emb1'], jnp.zeros((128 - 119, EMB), f32)], 0)
    emb2p = jnp.concatenate([p['x_emb2'], jnp.zeros((128 - 3, EMB), f32)], 0)

    h = _tc_embed(x0, x1, emb1p, emb2p)
    ctx = _tc_ctx(context, p['ce_W1'], row(p['ce_b1']),
                  p['ce_W2'], row(p['ce_b2']))

    cnt = _sc_segment_rows(ohrep2, code2, dst2, zerosE, 128)[:, :N, :16]

    for l in range(5):
        if l < 3:
            pre = 'basic%d_' % l
        else:
            pre = 'cf%d_' % (l - 3)
            gat2 = _tc_gat1(h, ctx, b2, p[pre + 'gat_Ws'], p[pre + 'gat_Wd'],
                            p[pre + 'gat_as'].reshape(EMB, 1),
                            p[pre + 'gat_ad'].reshape(EMB, 1),
                            row(p[pre + 'gat_b']))
            ctx, inj = _tc_gat2(ctx, gat2, p[pre + 'inj_W'],
                                row(p[pre + 'inj_b']))
            h = _tc_gat3(h, b2, inj)
        aggr = _sc_segment_rows(h, src2, dst2, zerosE, EMB)
        EE = jnp.concatenate(
            [p[pre + 'ee1'], p[pre + 'ee2'], jnp.zeros((8, EMB), f32)], 0)
        cself = (p[pre + 'ee1'][4] + p[pre + 'ee2'][0]).reshape(1, EMB)
        h = _tc_gine(aggr, h, cnt, EE, cself,
                     p[pre + 'W1'], row(p[pre + 'b1']),
                     p[pre + 'W2'], row(p[pre + 'b2']),
                     row(p[pre + 'bn_g']), row(p[pre + 'bn_b']),
                     do_relu=(l in (0, 1, 3)))

    return _tc_final(h, ctx, b2,
                     p['ot_W1'], row(p['ot_b1']), p['ot_W2'], row(p['ot_b2']),
                     p['rs_W'], row(p['rs_b']),
                     [p['mlp_W%d' % i] for i in range(4)],
                     [row(p['mlp_b%d' % i]) for i in range(4)])


# speedup vs baseline: 13.2086x; 1.2164x over previous
import functools

import jax
import jax.numpy as jnp
from jax import lax
from jax.experimental import pallas as pl
from jax.experimental.pallas import tpu as pltpu
from jax.experimental.pallas import tpu_sc as plsc

N = 10000
E = 320000
G = 256
EMB = 128
NPAD = 10240
NSUB = 16
CHUNK = 128
KW = 160
IB = 16
EPW = KW * CHUNK
EP = EPW * NSUB
CHUNKS = EP // CHUNK
ZR = NPAD // NSUB
REP = 128

HIGH = lax.Precision.HIGHEST


def _dot(a, b):
    return lax.dot_general(a, b, (((a.ndim - 1,), (0,)), ((), ())),
                           precision=HIGH, preferred_element_type=jnp.float32)


def _dotd(a, b):
    return lax.dot_general(a, b, (((a.ndim - 1,), (0,)), ((), ())),
                           preferred_element_type=jnp.float32)


def _dotT(a, b):
    return lax.dot_general(a, b, (((0,), (0,)), ((), ())),
                           precision=HIGH, preferred_element_type=jnp.float32)


def _sc_segment_rows(tab2, src2, dst2, zeros, d):
    mesh = plsc.VectorSubcoreMesh(core_axis_name="c", subcore_axis_name="s")

    @functools.partial(
        pl.kernel, mesh=mesh,
        out_type=jax.ShapeDtypeStruct((2, NPAD, d), jnp.float32),
        scratch_types=[
            pltpu.VMEM((IB * CHUNK,), jnp.int32),
            pltpu.VMEM((IB, CHUNK), jnp.int32),
            pltpu.VMEM((CHUNK, d), jnp.float32),
            pltpu.VMEM((CHUNK, d), jnp.float32),
            pltpu.VMEM_SHARED((NPAD, d), jnp.float32),
            pltpu.SemaphoreType.DMA,
            pltpu.SemaphoreType.DMA,
            pltpu.SemaphoreType.DMA,
            pltpu.SemaphoreType.DMA,
        ])
    def scatter_kernel(tab_h, src_h, dst_h, z_h, out_h,
                       src_v, dst_v, buf0, buf1, acc,
                       sem0, sem1, sem2, sem3):
        c = lax.axis_index("c")
        s = lax.axis_index("s")
        pltpu.sync_copy(z_h.at[pl.ds(s * ZR, ZR)], acc.at[pl.ds(s * ZR, ZR)])
        plsc.subcore_barrier()

        def run(tab, src, dst):
            def blk(ib, carry):
                pltpu.sync_copy(
                    src.at[pl.ds(s * EPW + ib * (IB * CHUNK), IB * CHUNK)],
                    src_v)
                pltpu.sync_copy(dst.at[pl.ds(s * KW + ib * IB, IB)], dst_v)

                bufs = (buf0, buf1)
                gsems = (sem0, sem1)
                ssems = (sem2, sem3)

                def gath(k, b):
                    return pltpu.async_copy(
                        tab.at[src_v.at[pl.ds(k * CHUNK, CHUNK)]],
                        bufs[b], gsems[b])

                g = [gath(0, 0), gath(1, 1)]
                sc = [None, None]
                for k in range(IB):
                    b = k % 2
                    g[b].wait()
                    sc[b] = pltpu.async_copy(bufs[b], acc.at[dst_v.at[k]],
                                             ssems[b], add=True)
                    if k + 2 < IB:
                        sc[b].wait()
                        g[b] = gath(k + 2, b)
                sc[0].wait()
                sc[1].wait()
                return carry

            lax.fori_loop(0, KW // IB, blk, 0)

        @pl.when(c == 0)
        def _():
            run(tab_h.at[0], src_h.at[0], dst_h.at[0])

        @pl.when(c == 1)
        def _():
            run(tab_h.at[1], src_h.at[1], dst_h.at[1])

        plsc.subcore_barrier()
        pltpu.sync_copy(acc.at[pl.ds(s * ZR, ZR)],
                        out_h.at[c].at[pl.ds(s * ZR, ZR)])

    return scatter_kernel(tab2, src2, dst2, zeros)


EMB_BLK = 2000


def _tc_embed(x0, x1, emb1p, emb2p):
    def body(x0_r, x1_r, e1_r, e2_r, h_r):
        oh1 = (x0_r[0] == lax.broadcasted_iota(jnp.int32, (EMB_BLK, 128), 1)
               ).astype(jnp.float32)
        oh2 = (x1_r[0] == lax.broadcasted_iota(jnp.int32, (EMB_BLK, 128), 1)
               ).astype(jnp.float32)
        h_r[0] = _dot(oh1, e1_r[...]) + _dot(oh2, e2_r[...])

    full = lambda shape: pl.BlockSpec(shape, lambda g, b: (0,) * len(shape))
    return pl.pallas_call(
        body, grid=(2, N // EMB_BLK),
        in_specs=[
            pl.BlockSpec((1, EMB_BLK, 1), lambda g, b: (g, b, 0)),
            pl.BlockSpec((1, EMB_BLK, 1), lambda g, b: (g, b, 0)),
            full((128, EMB)), full((128, EMB)),
        ],
        out_specs=pl.BlockSpec((1, EMB_BLK, EMB), lambda g, b: (g, b, 0)),
        out_shape=jax.ShapeDtypeStruct((2, N, EMB), jnp.float32),
    )(x0, x1, emb1p, emb2p)


def _tc_ctx(context, ceW1, ceb1, ceW2, ceb2):
    def body(ctx_r, w1_r, b1_r, w2_r, b2_r, c_r):
        t = jnp.maximum(_dotd(ctx_r[...], w1_r[...]) + b1_r[...], 0.0)
        c_r[...] = _dotd(t, w2_r[...]) + b2_r[...]

    return pl.pallas_call(
        body,
        out_shape=jax.ShapeDtypeStruct((G, EMB), jnp.float32),
    )(context, ceW1, ceb1, ceW2, ceb2)


GINE_T = 1000


def _tc_gine(aggr, h, cnt, EE, cself, W1, b1, W2, b2, bng, bnb, do_relu):
    def body(a_r, h_r, c_r, ee_r, cs_r, w1_r, b1_r, w2_r, b2_r, g_r, bb_r,
             o_r, t_s):
        nt = N // GINE_T

        def p1(i, s1):
            sl = pl.ds(i * GINE_T, GINE_T)
            a = (a_r[0, sl, :] + h_r[0, sl, :] + cs_r[...]
                 + _dot(c_r[0, sl, :], ee_r[...]))
            t = jnp.maximum(_dotd(a, w1_r[...]) + b1_r[...], 0.0)
            t = _dotd(t, w2_r[...]) + b2_r[...]
            t_s[sl, :] = t
            return s1 + jnp.sum(t, axis=0, keepdims=True)

        s1 = lax.fori_loop(0, nt, p1, jnp.zeros((1, EMB), jnp.float32))
        m = s1 * (1.0 / N)

        def p2(i, s2):
            sl = pl.ds(i * GINE_T, GINE_T)
            d = t_s[sl, :] - m
            return s2 + jnp.sum(d * d, axis=0, keepdims=True)

        s2 = lax.fori_loop(0, nt, p2, jnp.zeros((1, EMB), jnp.float32))
        v = s2 * (1.0 / N)
        sc = lax.rsqrt(v + 1e-5) * g_r[...]
        sh = bb_r[...] - m * sc

        def p3(i, c):
            sl = pl.ds(i * GINE_T, GINE_T)
            hn = t_s[sl, :] * sc + sh
            if do_relu:
                hn = jnp.maximum(hn, 0.0)
            o_r[0, sl, :] = hn
            return c

        lax.fori_loop(0, nt, p3, 0)

    full = lambda shape: pl.BlockSpec(shape, lambda g: (0,) * len(shape))
    return pl.pallas_call(
        body, grid=(2,),
        in_specs=[
            pl.BlockSpec((1, NPAD, EMB), lambda g: (g, 0, 0)),
            pl.BlockSpec((1, N, EMB), lambda g: (g, 0, 0)),
            pl.BlockSpec((1, N, 16), lambda g: (g, 0, 0)),
            full((16, EMB)), full((1, EMB)),
            full((EMB, 2 * EMB)), full((1, 2 * EMB)),
            full((2 * EMB, EMB)), full((1, EMB)),
            full((1, EMB)), full((1, EMB)),
        ],
        out_specs=pl.BlockSpec((1, N, EMB), lambda g: (g, 0, 0)),
        out_shape=jax.ShapeDtypeStruct((2, N, EMB), jnp.float32),
        scratch_shapes=[pltpu.VMEM((N, EMB), jnp.float32)],
    )(aggr, h, cnt, EE, cself, W1, b1, W2, b2, bng, bnb)


GAT_T = 1000


def _tc_gat1(h, ctx, b2, Ws, Wd, a_s, a_d, gatb):
    def body(h_r, ctx_r, b_r, ws_r, wd_r, as_r, ad_r, gb_r, o_r, hs_s, e_s):
        nt = N // GAT_T

        def onehot(i):
            sl = pl.ds(i * GAT_T, GAT_T)
            return (b_r[0, sl, :]
                    == lax.broadcasted_iota(jnp.int32, (GAT_T, G), 1)
                    ).astype(jnp.float32)

        hd = _dotd(ctx_r[...], wd_r[...])
        ed = _dotd(hd, ad_r[...])

        def p1(i, M):
            sl = pl.ds(i * GAT_T, GAT_T)
            O = onehot(i)
            hs = _dotd(h_r[0, sl, :], ws_r[...])
            hs_s[sl, :] = hs
            e = _dotd(hs, as_r[...]) + _dot(O, ed)
            e = jnp.where(e > 0, e, 0.2 * e)
            e_s[sl, :] = e
            return jnp.maximum(
                M, jnp.max(e - 1e30 * (1.0 - O), axis=0, keepdims=True))

        M = lax.fori_loop(0, nt, p1, jnp.full((1, G), -1e30, jnp.float32))
        Mc = jnp.reshape(M, (G, 1))

        def p2(i, ssum):
            sl = pl.ds(i * GAT_T, GAT_T)
            O = onehot(i)
            ex = jnp.exp(e_s[sl, :] - _dot(O, Mc))
            return ssum + _dotT(O, ex)

        ssum = lax.fori_loop(0, nt, p2, jnp.zeros((G, 1), jnp.float32))

        def p3(i, acc):
            sl = pl.ds(i * GAT_T, GAT_T)
            O = onehot(i)
            ex = jnp.exp(e_s[sl, :] - _dot(O, Mc))
            alpha = ex / (_dot(O, ssum) + 1e-16)
            return acc + _dotT(O, alpha * hs_s[sl, :])

        acc = lax.fori_loop(0, nt, p3, jnp.zeros((G, EMB), jnp.float32))
        o_r[0] = acc + gb_r[...]

    full = lambda shape: pl.BlockSpec(shape, lambda g: (0,) * len(shape))
    return pl.pallas_call(
        body, grid=(2,),
        in_specs=[
            pl.BlockSpec((1, N, EMB), lambda g: (g, 0, 0)),
            full((G, EMB)),
            pl.BlockSpec((1, N, 1), lambda g: (g, 0, 0)),
            full((EMB, EMB)), full((EMB, EMB)),
            full((EMB, 1)), full((EMB, 1)), full((1, EMB)),
        ],
        out_specs=pl.BlockSpec((1, G, EMB), lambda g: (g, 0, 0)),
        out_shape=jax.ShapeDtypeStruct((2, G, EMB), jnp.float32),
        scratch_shapes=[pltpu.VMEM((N, EMB), jnp.float32),
                        pltpu.VMEM((N, 1), jnp.float32)],
    )(h, ctx, b2, Ws, Wd, a_s, a_d, gatb)


def _tc_gat2(ctx, gat2, injW, injb):
    def body(ctx_r, g_r, w_r, b_r, c_r, i_r):
        cn = ctx_r[...] + g_r[0] + g_r[1]
        c_r[...] = cn
        i_r[...] = _dotd(cn, w_r[...]) + b_r[...]

    return pl.pallas_call(
        body,
        out_shape=(jax.ShapeDtypeStruct((G, EMB), jnp.float32),
                   jax.ShapeDtypeStruct((G, EMB), jnp.float32)),
    )(ctx, gat2, injW, injb)


def _tc_gat3(h, b2, inj):
    def body(h_r, b_r, i_r, o_r):
        O = (b_r[0] == lax.broadcasted_iota(jnp.int32, (GAT_T, G), 1)
             ).astype(jnp.float32)
        o_r[0] = h_r[0] + _dot(O, i_r[...])

    full = lambda shape: pl.BlockSpec(shape, lambda g, b: (0,) * len(shape))
    return pl.pallas_call(
        body, grid=(2, N // GAT_T),
        in_specs=[
            pl.BlockSpec((1, GAT_T, EMB), lambda g, b: (g, b, 0)),
            pl.BlockSpec((1, GAT_T, 1), lambda g, b: (g, b, 0)),
            full((G, EMB)),
        ],
        out_specs=pl.BlockSpec((1, GAT_T, EMB), lambda g, b: (g, b, 0)),
        out_shape=jax.ShapeDtypeStruct((2, N, EMB), jnp.float32),
    )(h, b2, inj)


def _tc_final(h, ctx, b2, otW1, otb1, otW2, otb2, rsW, rsb, mws, mbs):
    def body(h_r, ctx_r, b_r, ow1, ob1, ow2, ob2, rw, rb,
             m0, mb0, m1, mb1, m2, mb2, m3, mb3, o_r):
        gs = []
        for g in range(2):
            def pool(i, carry):
                acc, cnt = carry
                sl = pl.ds(i * GAT_T, GAT_T)
                O = (b_r[g, sl, :]
                     == lax.broadcasted_iota(jnp.int32, (GAT_T, G), 1)
                     ).astype(jnp.float32)
                return (acc + _dotT(O, h_r[g, sl, :]),
                        cnt + jnp.sum(O, axis=0, keepdims=True))

            acc, cnt = lax.fori_loop(
                0, N // GAT_T, pool,
                (jnp.zeros((G, EMB), jnp.float32),
                 jnp.zeros((1, G), jnp.float32)))
            cnt = jnp.maximum(cnt, 1.0)
            gm = acc / jnp.reshape(cnt, (G, 1))
            gm = jnp.maximum(_dotd(gm, ow1[...]) + ob1[...], 0.0)
            gs.append(_dotd(gm, ow2[...]) + ob2[...])
        r = _dotd(gs[0] + gs[1], rw[...]) + rb[...]
        z = jnp.concatenate([r, ctx_r[...]], axis=1)
        for w_r, b_r2 in ((m0, mb0), (m1, mb1), (m2, mb2)):
            z = _dotd(z, w_r[...]) + b_r2[...]
            z = jnp.where(z > 0, z, 0.01 * z)
        o_r[...] = _dotd(z, m3[...]) + mb3[...]

    return pl.pallas_call(
        body,
        out_shape=jax.ShapeDtypeStruct((G, 1), jnp.float32),
    )(h, ctx, b2, otW1, otb1, otW2, otb2, rsW, rsb,
      mws[0], mbs[0], mws[1], mbs[1], mws[2], mbs[2], mws[3], mbs[3])


def _prep_graph(ei, ea):
    pad = jnp.arange(EP - E, dtype=jnp.int32)
    src = jnp.concatenate([ei[0], pad % N])
    dst = jnp.concatenate([ei[1], N + pad % (NPAD - N)]).reshape(CHUNKS, CHUNK)
    ear = jnp.arange(E, dtype=jnp.int32)
    code = ea[:, 0] * 3 + ea[:, 1]
    codeidx = jnp.concatenate(
        [code * REP + ear % REP, (pad % 15) * REP + pad % REP])
    return src, dst, codeidx


def kernel(xA, edge_indexA, edge_attrA, batchA, xB, edge_indexB, edge_attrB,
           batchB, context, params):
    p = params
    f32 = jnp.float32
    row = lambda v: v.reshape(1, -1).astype(f32)

    srcA, dstA, codeA = _prep_graph(edge_indexA, edge_attrA)
    srcB, dstB, codeB = _prep_graph(edge_indexB, edge_attrB)
    src2 = jnp.stack([srcA, srcB])
    dst2 = jnp.stack([dstA, dstB])
    code2 = jnp.stack([codeA, codeB])
    b2 = jnp.stack([batchA, batchB]).reshape(2, N, 1)
    x0 = jnp.stack([xA[:, 0], xB[:, 0]]).reshape(2, N, 1)
    x1 = jnp.stack([xA[:, 1], xB[:, 1]]).reshape(2, N, 1)

    codes = jnp.arange(15, dtype=jnp.int32)
    lanes = jnp.arange(128, dtype=jnp.int32)
    oh = ((lanes[None, :] == codes[:, None] // 3)
          | (lanes[None, :] == 5 + codes[:, None] % 3)).astype(f32)
    ohrep = jnp.repeat(oh, REP, axis=0)
    ohrep2 = jnp.stack([ohrep, ohrep])

    zerosE = jnp.zeros((NPAD, EMB), f32)

    emb1p = jnp.concatenate([p['x_emb1'], jnp.zeros((128 - 119, EMB), f32)], 0)
    emb2p = jnp.concatenate([p['x_emb2'], jnp.zeros((128 - 3, EMB), f32)], 0)

    h = _tc_embed(x0, x1, emb1p, emb2p)
    ctx = _tc_ctx(context, p['ce_W1'], row(p['ce_b1']),
                  p['ce_W2'], row(p['ce_b2']))

    cnt = _sc_segment_rows(ohrep2, code2, dst2, zerosE, 128)[:, :N, :16]

    for l in range(5):
        if l < 3:
            pre = 'basic%d_' % l
        else:
            pre = 'cf%d_' % (l - 3)
            gat2 = _tc_gat1(h, ctx, b2, p[pre + 'gat_Ws'], p[pre + 'gat_Wd'],
                            p[pre + 'gat_as'].reshape(EMB, 1),
                            p[pre + 'gat_ad'].reshape(EMB, 1),
                            row(p[pre + 'gat_b']))
            ctx, inj = _tc_gat2(ctx, gat2, p[pre + 'inj_W'],
                                row(p[pre + 'inj_b']))
            h = _tc_gat3(h, b2, inj)
        aggr = _sc_segment_rows(h, src2, dst2, zerosE, EMB)
        EE = jnp.concatenate(
            [p[pre + 'ee1'], p[pre + 'ee2'], jnp.zeros((8, EMB), f32)], 0)
        cself = (p[pre + 'ee1'][4] + p[pre + 'ee2'][0]).reshape(1, EMB)
        h = _tc_gine(aggr, h, cnt, EE, cself,
                     p[pre + 'W1'], row(p[pre + 'b1']),
                     p[pre + 'W2'], row(p[pre + 'b2']),
                     row(p[pre + 'bn_g']), row(p[pre + 'bn_b']),
                     do_relu=(l in (0, 1, 3)))

    return _tc_final(h, ctx, b2,
                     p['ot_W1'], row(p['ot_b1']), p['ot_W2'], row(p['ot_b2']),
                     p['rs_W'], row(p['rs_b']),
                     [p['mlp_W%d' % i] for i in range(4)],
                     [row(p['mlp_b%d' % i]) for i in range(4)])


# speedup vs baseline: 13.6801x; 1.0357x over previous
import functools

import jax
import jax.numpy as jnp
from jax import lax
from jax.experimental import pallas as pl
from jax.experimental.pallas import tpu as pltpu
from jax.experimental.pallas import tpu_sc as plsc

N = 10000
E = 320000
G = 256
EMB = 128
NPAD = 10240
NSUB = 16
CHUNK = 128
KW = 160
IB = 32
EPW = KW * CHUNK
EP = EPW * NSUB
CHUNKS = EP // CHUNK
ZR = NPAD // NSUB
REP = 128

HIGH = lax.Precision.HIGHEST


def _dot(a, b):
    return lax.dot_general(a, b, (((a.ndim - 1,), (0,)), ((), ())),
                           precision=HIGH, preferred_element_type=jnp.float32)


def _dotd(a, b):
    return lax.dot_general(a, b, (((a.ndim - 1,), (0,)), ((), ())),
                           preferred_element_type=jnp.float32)


def _dotT(a, b):
    return lax.dot_general(a, b, (((0,), (0,)), ((), ())),
                           precision=HIGH, preferred_element_type=jnp.float32)


def _sc_segment_rows(tab2, src2, dst2, zeros, d):
    mesh = plsc.VectorSubcoreMesh(core_axis_name="c", subcore_axis_name="s")

    @functools.partial(
        pl.kernel, mesh=mesh,
        out_type=jax.ShapeDtypeStruct((2, NPAD, d), jnp.float32),
        scratch_types=[
            pltpu.VMEM((IB * CHUNK,), jnp.int32),
            pltpu.VMEM((IB, CHUNK), jnp.int32),
            pltpu.VMEM((CHUNK, d), jnp.float32),
            pltpu.VMEM((CHUNK, d), jnp.float32),
            pltpu.VMEM_SHARED((NPAD, d), jnp.float32),
            pltpu.SemaphoreType.DMA,
            pltpu.SemaphoreType.DMA,
            pltpu.SemaphoreType.DMA,
            pltpu.SemaphoreType.DMA,
        ])
    def scatter_kernel(tab_h, src_h, dst_h, z_h, out_h,
                       src_v, dst_v, buf0, buf1, acc,
                       sem0, sem1, sem2, sem3):
        c = lax.axis_index("c")
        s = lax.axis_index("s")
        pltpu.sync_copy(z_h.at[pl.ds(s * ZR, ZR)], acc.at[pl.ds(s * ZR, ZR)])
        plsc.subcore_barrier()

        def run(tab, src, dst):
            def blk(ib, carry):
                pltpu.sync_copy(
                    src.at[pl.ds(s * EPW + ib * (IB * CHUNK), IB * CHUNK)],
                    src_v)
                pltpu.sync_copy(dst.at[pl.ds(s * KW + ib * IB, IB)], dst_v)

                bufs = (buf0, buf1)
                gsems = (sem0, sem1)
                ssems = (sem2, sem3)

                def gath(k, b):
                    return pltpu.async_copy(
                        tab.at[src_v.at[pl.ds(k * CHUNK, CHUNK)]],
                        bufs[b], gsems[b])

                g = [gath(0, 0), gath(1, 1)]
                sc = [None, None]
                for k in range(IB):
                    b = k % 2
                    g[b].wait()
                    sc[b] = pltpu.async_copy(bufs[b], acc.at[dst_v.at[k]],
                                             ssems[b], add=True)
                    if k + 2 < IB:
                        sc[b].wait()
                        g[b] = gath(k + 2, b)
                sc[0].wait()
                sc[1].wait()
                return carry

            lax.fori_loop(0, KW // IB, blk, 0)

        @pl.when(c == 0)
        def _():
            run(tab_h.at[0], src_h.at[0], dst_h.at[0])

        @pl.when(c == 1)
        def _():
            run(tab_h.at[1], src_h.at[1], dst_h.at[1])

        plsc.subcore_barrier()
        pltpu.sync_copy(acc.at[pl.ds(s * ZR, ZR)],
                        out_h.at[c].at[pl.ds(s * ZR, ZR)])

    return scatter_kernel(tab2, src2, dst2, zeros)


EMB_BLK = 2000


def _tc_embed(x0, x1, emb1p, emb2p):
    def body(x0_r, x1_r, e1_r, e2_r, h_r):
        oh1 = (x0_r[0] == lax.broadcasted_iota(jnp.int32, (EMB_BLK, 128), 1)
               ).astype(jnp.float32)
        oh2 = (x1_r[0] == lax.broadcasted_iota(jnp.int32, (EMB_BLK, 128), 1)
               ).astype(jnp.float32)
        h_r[0] = _dot(oh1, e1_r[...]) + _dot(oh2, e2_r[...])

    full = lambda shape: pl.BlockSpec(shape, lambda g, b: (0,) * len(shape))
    return pl.pallas_call(
        body, grid=(2, N // EMB_BLK),
        in_specs=[
            pl.BlockSpec((1, EMB_BLK, 1), lambda g, b: (g, b, 0)),
            pl.BlockSpec((1, EMB_BLK, 1), lambda g, b: (g, b, 0)),
            full((128, EMB)), full((128, EMB)),
        ],
        out_specs=pl.BlockSpec((1, EMB_BLK, EMB), lambda g, b: (g, b, 0)),
        out_shape=jax.ShapeDtypeStruct((2, N, EMB), jnp.float32),
    )(x0, x1, emb1p, emb2p)


def _tc_ctx(context, ceW1, ceb1, ceW2, ceb2):
    def body(ctx_r, w1_r, b1_r, w2_r, b2_r, c_r):
        t = jnp.maximum(_dotd(ctx_r[...], w1_r[...]) + b1_r[...], 0.0)
        c_r[...] = _dotd(t, w2_r[...]) + b2_r[...]

    return pl.pallas_call(
        body,
        out_shape=jax.ShapeDtypeStruct((G, EMB), jnp.float32),
    )(context, ceW1, ceb1, ceW2, ceb2)


GINE_T = 1000


def _tc_gine(aggr, h, cnt, EE, cself, W1, b1, W2, b2, bng, bnb, do_relu):
    def body(a_r, h_r, c_r, ee_r, cs_r, w1_r, b1_r, w2_r, b2_r, g_r, bb_r,
             o_r, t_s):
        nt = N // GINE_T

        def p1(i, s1):
            sl = pl.ds(i * GINE_T, GINE_T)
            a = (a_r[0, sl, :] + h_r[0, sl, :] + cs_r[...]
                 + _dot(c_r[0, sl, :], ee_r[...]))
            t = jnp.maximum(_dotd(a, w1_r[...]) + b1_r[...], 0.0)
            t = _dotd(t, w2_r[...]) + b2_r[...]
            t_s[sl, :] = t
            return s1 + jnp.sum(t, axis=0, keepdims=True)

        s1 = lax.fori_loop(0, nt, p1, jnp.zeros((1, EMB), jnp.float32))
        m = s1 * (1.0 / N)

        def p2(i, s2):
            sl = pl.ds(i * GINE_T, GINE_T)
            d = t_s[sl, :] - m
            return s2 + jnp.sum(d * d, axis=0, keepdims=True)

        s2 = lax.fori_loop(0, nt, p2, jnp.zeros((1, EMB), jnp.float32))
        v = s2 * (1.0 / N)
        sc = lax.rsqrt(v + 1e-5) * g_r[...]
        sh = bb_r[...] - m * sc

        def p3(i, c):
            sl = pl.ds(i * GINE_T, GINE_T)
            hn = t_s[sl, :] * sc + sh
            if do_relu:
                hn = jnp.maximum(hn, 0.0)
            o_r[0, sl, :] = hn
            return c

        lax.fori_loop(0, nt, p3, 0)

    full = lambda shape: pl.BlockSpec(shape, lambda g: (0,) * len(shape))
    return pl.pallas_call(
        body, grid=(2,),
        in_specs=[
            pl.BlockSpec((1, NPAD, EMB), lambda g: (g, 0, 0)),
            pl.BlockSpec((1, N, EMB), lambda g: (g, 0, 0)),
            pl.BlockSpec((1, N, 16), lambda g: (g, 0, 0)),
            full((16, EMB)), full((1, EMB)),
            full((EMB, 2 * EMB)), full((1, 2 * EMB)),
            full((2 * EMB, EMB)), full((1, EMB)),
            full((1, EMB)), full((1, EMB)),
        ],
        out_specs=pl.BlockSpec((1, N, EMB), lambda g: (g, 0, 0)),
        out_shape=jax.ShapeDtypeStruct((2, N, EMB), jnp.float32),
        scratch_shapes=[pltpu.VMEM((N, EMB), jnp.float32)],
    )(aggr, h, cnt, EE, cself, W1, b1, W2, b2, bng, bnb)


GAT_T = 1000


def _tc_gat1(h, ctx, b2, Ws, Wd, a_s, a_d, gatb):
    def body(h_r, ctx_r, b_r, ws_r, wd_r, as_r, ad_r, gb_r, o_r, hs_s, e_s):
        nt = N // GAT_T

        def onehot(i):
            sl = pl.ds(i * GAT_T, GAT_T)
            return (b_r[0, sl, :]
                    == lax.broadcasted_iota(jnp.int32, (GAT_T, G), 1)
                    ).astype(jnp.float32)

        hd = _dotd(ctx_r[...], wd_r[...])
        ed = _dotd(hd, ad_r[...])

        def p1(i, M):
            sl = pl.ds(i * GAT_T, GAT_T)
            O = onehot(i)
            hs = _dotd(h_r[0, sl, :], ws_r[...])
            hs_s[sl, :] = hs
            e = _dotd(hs, as_r[...]) + _dot(O, ed)
            e = jnp.where(e > 0, e, 0.2 * e)
            e_s[sl, :] = e
            return jnp.maximum(
                M, jnp.max(e - 1e30 * (1.0 - O), axis=0, keepdims=True))

        M = lax.fori_loop(0, nt, p1, jnp.full((1, G), -1e30, jnp.float32))
        Mc = jnp.reshape(M, (G, 1))

        def p2(i, ssum):
            sl = pl.ds(i * GAT_T, GAT_T)
            O = onehot(i)
            ex = jnp.exp(e_s[sl, :] - _dot(O, Mc))
            return ssum + _dotT(O, ex)

        ssum = lax.fori_loop(0, nt, p2, jnp.zeros((G, 1), jnp.float32))

        def p3(i, acc):
            sl = pl.ds(i * GAT_T, GAT_T)
            O = onehot(i)
            ex = jnp.exp(e_s[sl, :] - _dot(O, Mc))
            alpha = ex / (_dot(O, ssum) + 1e-16)
            return acc + _dotT(O, alpha * hs_s[sl, :])

        acc = lax.fori_loop(0, nt, p3, jnp.zeros((G, EMB), jnp.float32))
        o_r[0] = acc + gb_r[...]

    full = lambda shape: pl.BlockSpec(shape, lambda g: (0,) * len(shape))
    return pl.pallas_call(
        body, grid=(2,),
        in_specs=[
            pl.BlockSpec((1, N, EMB), lambda g: (g, 0, 0)),
            full((G, EMB)),
            pl.BlockSpec((1, N, 1), lambda g: (g, 0, 0)),
            full((EMB, EMB)), full((EMB, EMB)),
            full((EMB, 1)), full((EMB, 1)), full((1, EMB)),
        ],
        out_specs=pl.BlockSpec((1, G, EMB), lambda g: (g, 0, 0)),
        out_shape=jax.ShapeDtypeStruct((2, G, EMB), jnp.float32),
        scratch_shapes=[pltpu.VMEM((N, EMB), jnp.float32),
                        pltpu.VMEM((N, 1), jnp.float32)],
    )(h, ctx, b2, Ws, Wd, a_s, a_d, gatb)


def _tc_gat2(ctx, gat2, injW, injb):
    def body(ctx_r, g_r, w_r, b_r, c_r, i_r):
        cn = ctx_r[...] + g_r[0] + g_r[1]
        c_r[...] = cn
        i_r[...] = _dotd(cn, w_r[...]) + b_r[...]

    return pl.pallas_call(
        body,
        out_shape=(jax.ShapeDtypeStruct((G, EMB), jnp.float32),
                   jax.ShapeDtypeStruct((G, EMB), jnp.float32)),
    )(ctx, gat2, injW, injb)


def _tc_gat3(h, b2, inj):
    def body(h_r, b_r, i_r, o_r):
        O = (b_r[0] == lax.broadcasted_iota(jnp.int32, (GAT_T, G), 1)
             ).astype(jnp.float32)
        o_r[0] = h_r[0] + _dot(O, i_r[...])

    full = lambda shape: pl.BlockSpec(shape, lambda g, b: (0,) * len(shape))
    return pl.pallas_call(
        body, grid=(2, N // GAT_T),
        in_specs=[
            pl.BlockSpec((1, GAT_T, EMB), lambda g, b: (g, b, 0)),
            pl.BlockSpec((1, GAT_T, 1), lambda g, b: (g, b, 0)),
            full((G, EMB)),
        ],
        out_specs=pl.BlockSpec((1, GAT_T, EMB), lambda g, b: (g, b, 0)),
        out_shape=jax.ShapeDtypeStruct((2, N, EMB), jnp.float32),
    )(h, b2, inj)


def _tc_final(h, ctx, b2, otW1, otb1, otW2, otb2, rsW, rsb, mws, mbs):
    def body(h_r, ctx_r, b_r, ow1, ob1, ow2, ob2, rw, rb,
             m0, mb0, m1, mb1, m2, mb2, m3, mb3, o_r):
        gs = []
        for g in range(2):
            def pool(i, carry):
                acc, cnt = carry
                sl = pl.ds(i * GAT_T, GAT_T)
                O = (b_r[g, sl, :]
                     == lax.broadcasted_iota(jnp.int32, (GAT_T, G), 1)
                     ).astype(jnp.float32)
                return (acc + _dotT(O, h_r[g, sl, :]),
                        cnt + jnp.sum(O, axis=0, keepdims=True))

            acc, cnt = lax.fori_loop(
                0, N // GAT_T, pool,
                (jnp.zeros((G, EMB), jnp.float32),
                 jnp.zeros((1, G), jnp.float32)))
            cnt = jnp.maximum(cnt, 1.0)
            gm = acc / jnp.reshape(cnt, (G, 1))
            gm = jnp.maximum(_dotd(gm, ow1[...]) + ob1[...], 0.0)
            gs.append(_dotd(gm, ow2[...]) + ob2[...])
        r = _dotd(gs[0] + gs[1], rw[...]) + rb[...]
        z = jnp.concatenate([r, ctx_r[...]], axis=1)
        for w_r, b_r2 in ((m0, mb0), (m1, mb1), (m2, mb2)):
            z = _dotd(z, w_r[...]) + b_r2[...]
            z = jnp.where(z > 0, z, 0.01 * z)
        o_r[...] = _dotd(z, m3[...]) + mb3[...]

    return pl.pallas_call(
        body,
        out_shape=jax.ShapeDtypeStruct((G, 1), jnp.float32),
    )(h, ctx, b2, otW1, otb1, otW2, otb2, rsW, rsb,
      mws[0], mbs[0], mws[1], mbs[1], mws[2], mbs[2], mws[3], mbs[3])


def _prep_graph(ei, ea):
    pad = jnp.arange(EP - E, dtype=jnp.int32)
    src = jnp.concatenate([ei[0], pad % N])
    dst = jnp.concatenate([ei[1], N + pad % (NPAD - N)]).reshape(CHUNKS, CHUNK)
    ear = jnp.arange(E, dtype=jnp.int32)
    code = ea[:, 0] * 3 + ea[:, 1]
    codeidx = jnp.concatenate(
        [code * REP + ear % REP, (pad % 15) * REP + pad % REP])
    return src, dst, codeidx


def kernel(xA, edge_indexA, edge_attrA, batchA, xB, edge_indexB, edge_attrB,
           batchB, context, params):
    p = params
    f32 = jnp.float32
    row = lambda v: v.reshape(1, -1).astype(f32)

    srcA, dstA, codeA = _prep_graph(edge_indexA, edge_attrA)
    srcB, dstB, codeB = _prep_graph(edge_indexB, edge_attrB)
    src2 = jnp.stack([srcA, srcB])
    dst2 = jnp.stack([dstA, dstB])
    code2 = jnp.stack([codeA, codeB])
    b2 = jnp.stack([batchA, batchB]).reshape(2, N, 1)
    x0 = jnp.stack([xA[:, 0], xB[:, 0]]).reshape(2, N, 1)
    x1 = jnp.stack([xA[:, 1], xB[:, 1]]).reshape(2, N, 1)

    codes = jnp.arange(15, dtype=jnp.int32)
    lanes = jnp.arange(128, dtype=jnp.int32)
    oh = ((lanes[None, :] == codes[:, None] // 3)
          | (lanes[None, :] == 5 + codes[:, None] % 3)).astype(f32)
    ohrep = jnp.repeat(oh, REP, axis=0)
    ohrep2 = jnp.stack([ohrep, ohrep])

    zerosE = jnp.zeros((NPAD, EMB), f32)

    emb1p = jnp.concatenate([p['x_emb1'], jnp.zeros((128 - 119, EMB), f32)], 0)
    emb2p = jnp.concatenate([p['x_emb2'], jnp.zeros((128 - 3, EMB), f32)], 0)

    h = _tc_embed(x0, x1, emb1p, emb2p)
    ctx = _tc_ctx(context, p['ce_W1'], row(p['ce_b1']),
                  p['ce_W2'], row(p['ce_b2']))

    cnt = _sc_segment_rows(ohrep2, code2, dst2, zerosE, 128)[:, :N, :16]

    for l in range(5):
        if l < 3:
            pre = 'basic%d_' % l
        else:
            pre = 'cf%d_' % (l - 3)
            gat2 = _tc_gat1(h, ctx, b2, p[pre + 'gat_Ws'], p[pre + 'gat_Wd'],
                            p[pre + 'gat_as'].reshape(EMB, 1),
                            p[pre + 'gat_ad'].reshape(EMB, 1),
                            row(p[pre + 'gat_b']))
            ctx, inj = _tc_gat2(ctx, gat2, p[pre + 'inj_W'],
                                row(p[pre + 'inj_b']))
            h = _tc_gat3(h, b2, inj)
        aggr = _sc_segment_rows(h, src2, dst2, zerosE, EMB)
        EE = jnp.concatenate(
            [p[pre + 'ee1'], p[pre + 'ee2'], jnp.zeros((8, EMB), f32)], 0)
        cself = (p[pre + 'ee1'][4] + p[pre + 'ee2'][0]).reshape(1, EMB)
        h = _tc_gine(aggr, h, cnt, EE, cself,
                     p[pre + 'W1'], row(p[pre + 'b1']),
                     p[pre + 'W2'], row(p[pre + 'b2']),
                     row(p[pre + 'bn_g']), row(p[pre + 'bn_b']),
                     do_relu=(l in (0, 1, 3)))

    return _tc_final(h, ctx, b2,
                     p['ot_W1'], row(p['ot_b1']), p['ot_W2'], row(p['ot_b2']),
                     p['rs_W'], row(p['rs_b']),
                     [p['mlp_W%d' % i] for i in range(4)],
                     [row(p['mlp_b%d' % i]) for i in range(4)])
